# Initial kernel scaffold; baseline (speedup 1.0000x reference)
import jax
import jax.numpy as jnp
from jax.experimental import pallas as pl


def kernel(x, edge_index, batch, W1, a_src1, a_dst1, b1, W2, a_src2, a_dst2, b2):
    raise NotImplementedError("write your pallas kernel here")


# speedup vs baseline: 11.2164x; 11.2164x over previous
import functools

import jax
import jax.numpy as jnp
from jax import lax
from jax.experimental import pallas as pl
from jax.experimental.pallas import tpu as pltpu
from jax.experimental.pallas import tpu_sc as plsc

N_NODES = 10000
N_PAD = 10240
IN_F = 128
HEADS = 8
HID = 64
NCH = 4
CH = 128
E_TOT = 170000
E_PAD = 172032
EPB = E_PAD // 32
BLK = 128
ABLK = 64
NG = 128
RB = 256

_MESH = plsc.VectorSubcoreMesh(core_axis_name="c", subcore_axis_name="s",
                               num_cores=2)
_ROWS = N_PAD // 16
N_ACC = 10112
_AROWS = N_ACC // 16


def _sc_alpha(tlog, srci, dsti, z16):
    @functools.partial(
        pl.kernel,
        out_type=[
            jax.ShapeDtypeStruct((2, N_ACC, CH), jnp.float32),
            jax.ShapeDtypeStruct((E_PAD, 16), jnp.float32),
        ],
        mesh=_MESH,
        scratch_types=[
            pltpu.VMEM((ABLK,), jnp.int32),
            pltpu.VMEM((ABLK,), jnp.int32),
            pltpu.VMEM((ABLK, CH), jnp.float32),
            pltpu.VMEM((ABLK, CH), jnp.float32),
            pltpu.VMEM((ABLK, CH), jnp.float32),
            pltpu.VMEM((ABLK, 16), jnp.float32),
            pltpu.VMEM_SHARED((N_ACC, CH), jnp.float32),
            pltpu.SemaphoreType.DMA,
        ],
    )
    def k(tlog_h, srci_h, dsti_h, z16_h, den_h, ea_h,
          sidx, didx, sbuf, dbuf, eabuf, eab16, den_sp, sem):
        c = lax.axis_index("c")
        s = lax.axis_index("s")
        pltpu.sync_copy(z16_h.at[pl.ds(s * _AROWS, _AROWS)],
                        den_sp.at[pl.ds(s * _AROWS, _AROWS)])
        pltpu.sync_copy(z16_h.at[pl.ds(0, ABLK)], eabuf)
        plsc.subcore_barrier()
        base0 = (c * 16 + s) * EPB

        @pl.loop(0, EPB // ABLK)
        def _(b):
            base = base0 + b * ABLK
            pltpu.sync_copy(srci_h.at[pl.ds(base, ABLK)], sidx)
            pltpu.sync_copy(dsti_h.at[pl.ds(base, ABLK)], didx)
            pltpu.async_copy(tlog_h.at[sidx], sbuf, sem).wait()
            pltpu.async_copy(tlog_h.at[didx], dbuf, sem).wait()

            @pl.loop(0, ABLK)
            def _(e):
                a = sbuf[e, pl.ds(0, 16)] + dbuf[e, pl.ds(16, 16)]
                ea = jnp.exp(jnp.maximum(a, 0.2 * a))
                eabuf[e, pl.ds(0, 16)] = ea
                eab16[e, :] = ea

            pltpu.sync_copy(eabuf, den_sp.at[didx], add=True)
            pltpu.sync_copy(eab16, ea_h.at[pl.ds(base, ABLK)])

        plsc.subcore_barrier()
        pltpu.sync_copy(den_sp.at[pl.ds(s * _AROWS, _AROWS)],
                        den_h.at[c].at[pl.ds(s * _AROWS, _AROWS)])

    return k(tlog, srci, dsti, z16)


def _sc_accum(h_tab, srci, dsti, ea, z128, lane0):
    @functools.partial(
        pl.kernel,
        out_type=jax.ShapeDtypeStruct((2, N_PAD, CH), jnp.float32),
        mesh=_MESH,
        scratch_types=[
            pltpu.VMEM((BLK,), jnp.int32),
            pltpu.VMEM((BLK,), jnp.int32),
            pltpu.VMEM((BLK, 16), jnp.float32),
            pltpu.VMEM((BLK, CH), jnp.float32),
            pltpu.VMEM((BLK, CH), jnp.float32),
            pltpu.VMEM_SHARED((N_ACC, CH), jnp.float32),
            pltpu.SemaphoreType.DMA,
        ],
    )
    def k(h_h, srci_h, dsti_h, ea_h, z_h, out_h,
          sidx, didx, eab, hb, wb, acc_sp, sem):
        c = lax.axis_index("c")
        s = lax.axis_index("s")
        pltpu.sync_copy(z_h.at[pl.ds(s * _AROWS, _AROWS)],
                        acc_sp.at[pl.ds(s * _AROWS, _AROWS)])
        plsc.subcore_barrier()
        base0 = (c * 16 + s) * EPB

        @pl.loop(0, EPB // BLK)
        def _(b):
            base = base0 + b * BLK
            pltpu.sync_copy(srci_h.at[pl.ds(base, BLK)], sidx)
            pltpu.sync_copy(dsti_h.at[pl.ds(base, BLK)], didx)
            pltpu.sync_copy(ea_h.at[pl.ds(base, BLK)], eab)
            pltpu.async_copy(h_h.at[sidx], hb, sem).wait()

            @pl.loop(0, BLK)
            def _(e):
                coef = eab[e, :]
                c0 = coef[lane0]
                c1 = coef[lane0 + 1]
                for j in range(4):
                    wb[e, pl.ds(j * 16, 16)] = hb[e, pl.ds(j * 16, 16)] * c0
                for j in range(4, 8):
                    wb[e, pl.ds(j * 16, 16)] = hb[e, pl.ds(j * 16, 16)] * c1

            pltpu.sync_copy(wb, acc_sp.at[didx], add=True)

        plsc.subcore_barrier()
        pltpu.sync_copy(acc_sp.at[pl.ds(s * _AROWS, _AROWS)],
                        out_h.at[c].at[pl.ds(s * _AROWS, _AROWS)])

        @pl.when(s == 0)
        def _():
            pltpu.sync_copy(z_h.at[pl.ds(0, N_PAD - N_ACC)], hb)
            pltpu.sync_copy(hb, out_h.at[c].at[pl.ds(N_ACC, N_PAD - N_ACC)])

    return k(h_tab, srci, dsti, ea, z128)


def _logit_tail(acc, asr_ref, adr_ref, tl_ref, c):
    a0 = asr_ref[0, 0]
    a1 = asr_ref[0, 1]
    d0 = adr_ref[0, 0]
    d1 = adr_ref[0, 1]
    s0 = jnp.sum(acc[:, :HID] * a0[None, :], axis=1)
    s1 = jnp.sum(acc[:, HID:] * a1[None, :], axis=1)
    t0 = jnp.sum(acc[:, :HID] * d0[None, :], axis=1)
    t1 = jnp.sum(acc[:, HID:] * d1[None, :], axis=1)
    col = lax.broadcasted_iota(jnp.int32, (RB, CH), 1)
    c2 = 2 * c

    @pl.when(c == 0)
    def _():
        tl_ref[...] = jnp.zeros_like(tl_ref)

    tl_ref[...] += (
        jnp.where((col == c2) | (col == c2 + 8), s0[:, None], 0.0)
        + jnp.where((col == c2 + 1) | (col == c2 + 9), s1[:, None], 0.0)
        + jnp.where((col == c2 + 16) | (col == c2 + 24), t0[:, None], 0.0)
        + jnp.where((col == c2 + 17) | (col == c2 + 25), t1[:, None], 0.0)
    )


def _tc_layer1(x_pad, W1, a_src, a_dst):
    def body(x_ref, w_ref, asr_ref, adr_ref, h_ref, tl_ref):
        c = pl.program_id(1)
        acc = jnp.dot(x_ref[...], w_ref[...], preferred_element_type=jnp.float32)
        h_ref[...] = acc[None]
        _logit_tail(acc, asr_ref, adr_ref, tl_ref, c)

    return pl.pallas_call(
        body,
        grid=(N_PAD // RB, NCH),
        in_specs=[
            pl.BlockSpec((RB, IN_F), lambda i, c: (i, 0)),
            pl.BlockSpec((IN_F, CH), lambda i, c: (0, c)),
            pl.BlockSpec((1, 2, HID), lambda i, c: (c, 0, 0)),
            pl.BlockSpec((1, 2, HID), lambda i, c: (c, 0, 0)),
        ],
        out_specs=[
            pl.BlockSpec((1, RB, CH), lambda i, c: (c, i, 0)),
            pl.BlockSpec((RB, CH), lambda i, c: (i, 0)),
        ],
        out_shape=[
            jax.ShapeDtypeStruct((NCH, N_PAD, CH), jnp.float32),
            jax.ShapeDtypeStruct((N_PAD, CH), jnp.float32),
        ],
    )(x_pad, W1, a_src.reshape(NCH, 2, HID), a_dst.reshape(NCH, 2, HID))


def _tc_layer2(p_list, inv1, b1, W2, a_src, a_dst):
    def body(p0, p1, p2, p3, inv_ref, b_ref, w_ref, asr_ref, adr_ref,
             h_ref, tl_ref):
        c = pl.program_id(1)
        parts = []
        for cb, p in enumerate((p0, p1, p2, p3)):
            i0 = jnp.broadcast_to(inv_ref[:, 2 * cb:2 * cb + 1], (RB, HID))
            i1 = jnp.broadcast_to(inv_ref[:, 2 * cb + 1:2 * cb + 2], (RB, HID))
            v = (p[0] + p[1]) * jnp.concatenate([i0, i1], axis=1) \
                + b_ref[cb][None, :]
            parts.append(jnp.where(v > 0, v, jnp.exp(v) - 1.0))
        hin = jnp.concatenate(parts, axis=1)
        acc = jnp.dot(hin, w_ref[...], preferred_element_type=jnp.float32)
        h_ref[...] = acc[None]
        _logit_tail(acc, asr_ref, adr_ref, tl_ref, c)

    p_spec = pl.BlockSpec((2, RB, CH), lambda i, c: (0, i, 0))
    return pl.pallas_call(
        body,
        grid=(N_PAD // RB, NCH),
        in_specs=[
            p_spec, p_spec, p_spec, p_spec,
            pl.BlockSpec((RB, 16), lambda i, c: (i, 0)),
            pl.BlockSpec((NCH, CH), lambda i, c: (0, 0)),
            pl.BlockSpec((HEADS * HID, CH), lambda i, c: (0, c)),
            pl.BlockSpec((1, 2, HID), lambda i, c: (c, 0, 0)),
            pl.BlockSpec((1, 2, HID), lambda i, c: (c, 0, 0)),
        ],
        out_specs=[
            pl.BlockSpec((1, RB, CH), lambda i, c: (c, i, 0)),
            pl.BlockSpec((RB, CH), lambda i, c: (i, 0)),
        ],
        out_shape=[
            jax.ShapeDtypeStruct((NCH, N_PAD, CH), jnp.float32),
            jax.ShapeDtypeStruct((N_PAD, CH), jnp.float32),
        ],
    )(*p_list, inv1, b1.reshape(NCH, CH), W2,
      a_src.reshape(NCH, 2, HID), a_dst.reshape(NCH, 2, HID))


def _tc_inv(den):
    def body(d_ref, o_ref):
        o_ref[...] = 1.0 / (d_ref[0, :, :16] + d_ref[1, :, :16] + 1e-16)

    out = pl.pallas_call(
        body,
        out_shape=jax.ShapeDtypeStruct((N_ACC, 16), jnp.float32),
    )(den)
    return jnp.zeros((N_PAD, 16), jnp.float32).at[:N_ACC].set(out)


def _tc_pool(q_list, inv2, b2, batch3):
    nsteps = N_PAD // RB

    def body(q0, q1, q2, q3, inv_ref, b_ref, b2_ref, out_ref, sums, cnt):
        i = pl.program_id(0)
        m = jnp.zeros((RB, HID), jnp.float32)
        for c, q in enumerate((q0, q1, q2, q3)):
            i0 = jnp.broadcast_to(inv_ref[:, 2 * c:2 * c + 1], (RB, HID))
            i1 = jnp.broadcast_to(inv_ref[:, 2 * c + 1:2 * c + 2], (RB, HID))
            m = m + (q[0, :, :HID] + q[1, :, :HID]) * i0 \
                  + (q[0, :, HID:] + q[1, :, HID:]) * i1
        o = m * (1.0 / HEADS)
        bv = b_ref[0, 0]
        gi = lax.broadcasted_iota(jnp.int32, (NG, RB), 0)
        oh = (bv[None, :] == gi).astype(jnp.float32)

        @pl.when(i == 0)
        def _():
            sums[...] = jnp.zeros_like(sums)
            cnt[...] = jnp.zeros_like(cnt)

        sums[...] += jnp.dot(oh, o, preferred_element_type=jnp.float32)
        cnt[...] += jnp.dot(oh, jnp.ones((RB, HID), jnp.float32),
                            preferred_element_type=jnp.float32)

        @pl.when(i == nsteps - 1)
        def _():
            out_ref[...] = (sums[...] / jnp.maximum(cnt[...], 1.0)
                            + jnp.where(cnt[...] > 0, 1.0, 0.0)
                            * b2_ref[0][None, :])

    q_spec = pl.BlockSpec((2, RB, CH), lambda i: (0, i, 0))
    return pl.pallas_call(
        body,
        grid=(nsteps,),
        in_specs=[
            q_spec, q_spec, q_spec, q_spec,
            pl.BlockSpec((RB, 16), lambda i: (i, 0)),
            pl.BlockSpec((1, 1, RB), lambda i: (i, 0, 0)),
            pl.BlockSpec((1, HID), lambda i: (0, 0)),
        ],
        out_specs=pl.BlockSpec((NG, HID), lambda i: (0, 0)),
        out_shape=jax.ShapeDtypeStruct((NG, HID), jnp.float32),
        scratch_shapes=[
            pltpu.VMEM((NG, HID), jnp.float32),
            pltpu.VMEM((NG, HID), jnp.float32),
        ],
    )(*q_list, inv2, batch3, b2.reshape(1, HID))


def kernel(x, edge_index, batch, W1, a_src1, a_dst1, b1, W2, a_src2, a_dst2, b2):
    x_pad = jnp.zeros((N_PAD, IN_F), jnp.float32).at[:N_NODES].set(x)
    loops = jnp.arange(N_NODES, dtype=jnp.int32)
    pad_i = jnp.full((E_PAD - E_TOT,), N_NODES, jnp.int32)
    srci = jnp.concatenate([edge_index[0], loops, pad_i])
    dsti = jnp.concatenate([edge_index[1], loops, pad_i])
    batch3 = jnp.concatenate(
        [batch, jnp.full((N_PAD - N_NODES,), -1, jnp.int32)]
    ).reshape(N_PAD // RB, 1, RB)
    z128 = jnp.zeros((N_PAD, CH), jnp.float32)

    h1, tl1 = _tc_layer1(x_pad, W1, a_src1, a_dst1)
    den1, ea1 = _sc_alpha(tl1, srci, dsti, z128)
    inv1 = _tc_inv(den1)
    p = [_sc_accum(h1[cid], srci, dsti, ea1, z128, 2 * cid)
         for cid in range(NCH)]

    h2, tl2 = _tc_layer2(p, inv1, b1, W2, a_src2, a_dst2)
    den2, ea2 = _sc_alpha(tl2, srci, dsti, z128)
    inv2 = _tc_inv(den2)
    q = [_sc_accum(h2[cid], srci, dsti, ea2, z128, 2 * cid)
         for cid in range(NCH)]

    return _tc_pool(q, inv2, b2, batch3)


# speedup vs baseline: 13.1329x; 1.1709x over previous
import functools

import jax
import jax.numpy as jnp
from jax import lax
from jax.experimental import pallas as pl
from jax.experimental.pallas import tpu as pltpu
from jax.experimental.pallas import tpu_sc as plsc

N_NODES = 10000
N_PAD = 10240
IN_F = 128
HEADS = 8
HID = 64
NCH = 4
CH = 128
E_TOT = 170000
E_PAD = 172032
EPB = E_PAD // 32
BLK = 128
ABLK = 64
NG = 128
RB = 256

_MESH = plsc.VectorSubcoreMesh(core_axis_name="c", subcore_axis_name="s",
                               num_cores=2)
_ROWS = N_PAD // 16
N_ACC = 10112
_AROWS = N_ACC // 16


def _sc_alpha(tlog, srci, dsti, z16):
    @functools.partial(
        pl.kernel,
        out_type=[
            jax.ShapeDtypeStruct((2, N_ACC, CH), jnp.float32),
            jax.ShapeDtypeStruct((E_PAD, 16), jnp.float32),
        ],
        mesh=_MESH,
        scratch_types=[
            pltpu.VMEM((ABLK,), jnp.int32),
            pltpu.VMEM((ABLK,), jnp.int32),
            pltpu.VMEM((ABLK, CH), jnp.float32),
            pltpu.VMEM((ABLK, CH), jnp.float32),
            pltpu.VMEM((ABLK, CH), jnp.float32),
            pltpu.VMEM((ABLK, 16), jnp.float32),
            pltpu.VMEM_SHARED((N_ACC, CH), jnp.float32),
            pltpu.SemaphoreType.DMA,
        ],
    )
    def k(tlog_h, srci_h, dsti_h, z16_h, den_h, ea_h,
          sidx, didx, sbuf, dbuf, eabuf, eab16, den_sp, sem):
        c = lax.axis_index("c")
        s = lax.axis_index("s")
        pltpu.sync_copy(z16_h.at[pl.ds(s * _AROWS, _AROWS)],
                        den_sp.at[pl.ds(s * _AROWS, _AROWS)])
        pltpu.sync_copy(z16_h.at[pl.ds(0, ABLK)], eabuf)
        plsc.subcore_barrier()
        base0 = (c * 16 + s) * EPB

        @pl.loop(0, EPB // ABLK)
        def _(b):
            base = base0 + b * ABLK
            cp_s = pltpu.async_copy(srci_h.at[pl.ds(base, ABLK)], sidx, sem)
            cp_d = pltpu.async_copy(dsti_h.at[pl.ds(base, ABLK)], didx, sem)
            cp_s.wait()
            cp_d.wait()
            cp_g1 = pltpu.async_copy(tlog_h.at[sidx], sbuf, sem)
            cp_g2 = pltpu.async_copy(tlog_h.at[didx], dbuf, sem)
            cp_g1.wait()
            cp_g2.wait()

            @pl.loop(0, ABLK)
            def _(e):
                a = sbuf[e, pl.ds(0, 16)] + dbuf[e, pl.ds(16, 16)]
                ea = jnp.exp(jnp.maximum(a, 0.2 * a))
                eabuf[e, pl.ds(0, 16)] = ea
                eab16[e, :] = ea

            pltpu.sync_copy(eabuf, den_sp.at[didx], add=True)
            pltpu.sync_copy(eab16, ea_h.at[pl.ds(base, ABLK)])

        plsc.subcore_barrier()
        pltpu.sync_copy(den_sp.at[pl.ds(s * _AROWS, _AROWS)],
                        den_h.at[c].at[pl.ds(s * _AROWS, _AROWS)])

    return k(tlog, srci, dsti, z16)


def _sc_accum(h_tab, srci, dsti, ea, z128, lane0):
    @functools.partial(
        pl.kernel,
        out_type=jax.ShapeDtypeStruct((2, N_PAD, CH), jnp.float32),
        mesh=_MESH,
        scratch_types=[
            pltpu.VMEM((BLK,), jnp.int32),
            pltpu.VMEM((BLK,), jnp.int32),
            pltpu.VMEM((BLK, 16), jnp.float32),
            pltpu.VMEM((BLK, CH), jnp.float32),
            pltpu.VMEM((BLK, CH), jnp.float32),
            pltpu.VMEM_SHARED((N_ACC, CH), jnp.float32),
            pltpu.SemaphoreType.DMA,
        ],
    )
    def k(h_h, srci_h, dsti_h, ea_h, z_h, out_h,
          sidx, didx, eab, hb, wb, acc_sp, sem):
        c = lax.axis_index("c")
        s = lax.axis_index("s")
        pltpu.sync_copy(z_h.at[pl.ds(s * _AROWS, _AROWS)],
                        acc_sp.at[pl.ds(s * _AROWS, _AROWS)])
        plsc.subcore_barrier()
        base0 = (c * 16 + s) * EPB

        @pl.loop(0, EPB // BLK)
        def _(b):
            base = base0 + b * BLK
            cp_s = pltpu.async_copy(srci_h.at[pl.ds(base, BLK)], sidx, sem)
            cp_d = pltpu.async_copy(dsti_h.at[pl.ds(base, BLK)], didx, sem)
            cp_e = pltpu.async_copy(ea_h.at[pl.ds(base, BLK)], eab, sem)
            cp_s.wait()
            cp_g = pltpu.async_copy(h_h.at[sidx], hb, sem)
            cp_d.wait()
            cp_e.wait()
            cp_g.wait()

            @pl.loop(0, BLK)
            def _(e):
                coef = eab[e, :]
                c0 = coef[lane0]
                c1 = coef[lane0 + 1]
                for j in range(4):
                    wb[e, pl.ds(j * 16, 16)] = hb[e, pl.ds(j * 16, 16)] * c0
                for j in range(4, 8):
                    wb[e, pl.ds(j * 16, 16)] = hb[e, pl.ds(j * 16, 16)] * c1

            pltpu.sync_copy(wb, acc_sp.at[didx], add=True)

        plsc.subcore_barrier()
        pltpu.sync_copy(acc_sp.at[pl.ds(s * _AROWS, _AROWS)],
                        out_h.at[c].at[pl.ds(s * _AROWS, _AROWS)])

        @pl.when(s == 0)
        def _():
            pltpu.sync_copy(z_h.at[pl.ds(0, N_PAD - N_ACC)], hb)
            pltpu.sync_copy(hb, out_h.at[c].at[pl.ds(N_ACC, N_PAD - N_ACC)])

    return k(h_tab, srci, dsti, ea, z128)


def _logit_tail(acc, asr_ref, adr_ref, tl_ref, c):
    a0 = asr_ref[0, 0]
    a1 = asr_ref[0, 1]
    d0 = adr_ref[0, 0]
    d1 = adr_ref[0, 1]
    s0 = jnp.sum(acc[:, :HID] * a0[None, :], axis=1)
    s1 = jnp.sum(acc[:, HID:] * a1[None, :], axis=1)
    t0 = jnp.sum(acc[:, :HID] * d0[None, :], axis=1)
    t1 = jnp.sum(acc[:, HID:] * d1[None, :], axis=1)
    col = lax.broadcasted_iota(jnp.int32, (RB, CH), 1)
    c2 = 2 * c

    @pl.when(c == 0)
    def _():
        tl_ref[...] = jnp.zeros_like(tl_ref)

    tl_ref[...] += (
        jnp.where((col == c2) | (col == c2 + 8), s0[:, None], 0.0)
        + jnp.where((col == c2 + 1) | (col == c2 + 9), s1[:, None], 0.0)
        + jnp.where((col == c2 + 16) | (col == c2 + 24), t0[:, None], 0.0)
        + jnp.where((col == c2 + 17) | (col == c2 + 25), t1[:, None], 0.0)
    )


def _tc_layer1(x_pad, W1, a_src, a_dst):
    def body(x_ref, w_ref, asr_ref, adr_ref, h_ref, tl_ref):
        c = pl.program_id(1)
        acc = jnp.dot(x_ref[...], w_ref[...], preferred_element_type=jnp.float32)
        h_ref[...] = acc[None]
        _logit_tail(acc, asr_ref, adr_ref, tl_ref, c)

    return pl.pallas_call(
        body,
        grid=(N_PAD // RB, NCH),
        in_specs=[
            pl.BlockSpec((RB, IN_F), lambda i, c: (i, 0)),
            pl.BlockSpec((IN_F, CH), lambda i, c: (0, c)),
            pl.BlockSpec((1, 2, HID), lambda i, c: (c, 0, 0)),
            pl.BlockSpec((1, 2, HID), lambda i, c: (c, 0, 0)),
        ],
        out_specs=[
            pl.BlockSpec((1, RB, CH), lambda i, c: (c, i, 0)),
            pl.BlockSpec((RB, CH), lambda i, c: (i, 0)),
        ],
        out_shape=[
            jax.ShapeDtypeStruct((NCH, N_PAD, CH), jnp.float32),
            jax.ShapeDtypeStruct((N_PAD, CH), jnp.float32),
        ],
    )(x_pad, W1, a_src.reshape(NCH, 2, HID), a_dst.reshape(NCH, 2, HID))


def _tc_layer2(p_list, inv1, b1, W2, a_src, a_dst):
    def body(p0, p1, p2, p3, inv_ref, b_ref, w_ref, asr_ref, adr_ref,
             h_ref, tl_ref):
        c = pl.program_id(1)
        parts = []
        for cb, p in enumerate((p0, p1, p2, p3)):
            i0 = jnp.broadcast_to(inv_ref[:, 2 * cb:2 * cb + 1], (RB, HID))
            i1 = jnp.broadcast_to(inv_ref[:, 2 * cb + 1:2 * cb + 2], (RB, HID))
            v = (p[0] + p[1]) * jnp.concatenate([i0, i1], axis=1) \
                + b_ref[cb][None, :]
            parts.append(jnp.where(v > 0, v, jnp.exp(v) - 1.0))
        hin = jnp.concatenate(parts, axis=1)
        acc = jnp.dot(hin, w_ref[...], preferred_element_type=jnp.float32)
        h_ref[...] = acc[None]
        _logit_tail(acc, asr_ref, adr_ref, tl_ref, c)

    p_spec = pl.BlockSpec((2, RB, CH), lambda i, c: (0, i, 0))
    return pl.pallas_call(
        body,
        grid=(N_PAD // RB, NCH),
        in_specs=[
            p_spec, p_spec, p_spec, p_spec,
            pl.BlockSpec((RB, 16), lambda i, c: (i, 0)),
            pl.BlockSpec((NCH, CH), lambda i, c: (0, 0)),
            pl.BlockSpec((HEADS * HID, CH), lambda i, c: (0, c)),
            pl.BlockSpec((1, 2, HID), lambda i, c: (c, 0, 0)),
            pl.BlockSpec((1, 2, HID), lambda i, c: (c, 0, 0)),
        ],
        out_specs=[
            pl.BlockSpec((1, RB, CH), lambda i, c: (c, i, 0)),
            pl.BlockSpec((RB, CH), lambda i, c: (i, 0)),
        ],
        out_shape=[
            jax.ShapeDtypeStruct((NCH, N_PAD, CH), jnp.float32),
            jax.ShapeDtypeStruct((N_PAD, CH), jnp.float32),
        ],
    )(*p_list, inv1, b1.reshape(NCH, CH), W2,
      a_src.reshape(NCH, 2, HID), a_dst.reshape(NCH, 2, HID))


def _tc_inv(den):
    def body(d_ref, o_ref):
        o_ref[...] = 1.0 / (d_ref[0, :, :16] + d_ref[1, :, :16] + 1e-16)

    out = pl.pallas_call(
        body,
        out_shape=jax.ShapeDtypeStruct((N_ACC, 16), jnp.float32),
    )(den)
    return jnp.zeros((N_PAD, 16), jnp.float32).at[:N_ACC].set(out)


def _tc_pool(q_list, inv2, b2, batch3):
    nsteps = N_PAD // RB

    def body(q0, q1, q2, q3, inv_ref, b_ref, b2_ref, out_ref, sums, cnt):
        i = pl.program_id(0)
        m = jnp.zeros((RB, HID), jnp.float32)
        for c, q in enumerate((q0, q1, q2, q3)):
            i0 = jnp.broadcast_to(inv_ref[:, 2 * c:2 * c + 1], (RB, HID))
            i1 = jnp.broadcast_to(inv_ref[:, 2 * c + 1:2 * c + 2], (RB, HID))
            m = m + (q[0, :, :HID] + q[1, :, :HID]) * i0 \
                  + (q[0, :, HID:] + q[1, :, HID:]) * i1
        o = m * (1.0 / HEADS)
        bv = b_ref[0, 0]
        gi = lax.broadcasted_iota(jnp.int32, (NG, RB), 0)
        oh = (bv[None, :] == gi).astype(jnp.float32)

        @pl.when(i == 0)
        def _():
            sums[...] = jnp.zeros_like(sums)
            cnt[...] = jnp.zeros_like(cnt)

        sums[...] += jnp.dot(oh, o, preferred_element_type=jnp.float32)
        cnt[...] += jnp.dot(oh, jnp.ones((RB, HID), jnp.float32),
                            preferred_element_type=jnp.float32)

        @pl.when(i == nsteps - 1)
        def _():
            out_ref[...] = (sums[...] / jnp.maximum(cnt[...], 1.0)
                            + jnp.where(cnt[...] > 0, 1.0, 0.0)
                            * b2_ref[0][None, :])

    q_spec = pl.BlockSpec((2, RB, CH), lambda i: (0, i, 0))
    return pl.pallas_call(
        body,
        grid=(nsteps,),
        in_specs=[
            q_spec, q_spec, q_spec, q_spec,
            pl.BlockSpec((RB, 16), lambda i: (i, 0)),
            pl.BlockSpec((1, 1, RB), lambda i: (i, 0, 0)),
            pl.BlockSpec((1, HID), lambda i: (0, 0)),
        ],
        out_specs=pl.BlockSpec((NG, HID), lambda i: (0, 0)),
        out_shape=jax.ShapeDtypeStruct((NG, HID), jnp.float32),
        scratch_shapes=[
            pltpu.VMEM((NG, HID), jnp.float32),
            pltpu.VMEM((NG, HID), jnp.float32),
        ],
    )(*q_list, inv2, batch3, b2.reshape(1, HID))


def kernel(x, edge_index, batch, W1, a_src1, a_dst1, b1, W2, a_src2, a_dst2, b2):
    x_pad = jnp.zeros((N_PAD, IN_F), jnp.float32).at[:N_NODES].set(x)
    loops = jnp.arange(N_NODES, dtype=jnp.int32)
    pad_i = jnp.full((E_PAD - E_TOT,), N_NODES, jnp.int32)
    srci = jnp.concatenate([edge_index[0], loops, pad_i])
    dsti = jnp.concatenate([edge_index[1], loops, pad_i])
    batch3 = jnp.concatenate(
        [batch, jnp.full((N_PAD - N_NODES,), -1, jnp.int32)]
    ).reshape(N_PAD // RB, 1, RB)
    z128 = jnp.zeros((N_PAD, CH), jnp.float32)

    h1, tl1 = _tc_layer1(x_pad, W1, a_src1, a_dst1)
    den1, ea1 = _sc_alpha(tl1, srci, dsti, z128)
    inv1 = _tc_inv(den1)
    p = [_sc_accum(h1[cid], srci, dsti, ea1, z128, 2 * cid)
         for cid in range(NCH)]

    h2, tl2 = _tc_layer2(p, inv1, b1, W2, a_src2, a_dst2)
    den2, ea2 = _sc_alpha(tl2, srci, dsti, z128)
    inv2 = _tc_inv(den2)
    q = [_sc_accum(h2[cid], srci, dsti, ea2, z128, 2 * cid)
         for cid in range(NCH)]

    return _tc_pool(q, inv2, b2, batch3)
